# Initial kernel scaffold; baseline (speedup 1.0000x reference)
import jax
import jax.numpy as jnp
from jax.experimental import pallas as pl


def kernel(ray, grid):
    raise NotImplementedError("write your pallas kernel here")


# speedup vs baseline: 1.6836x; 1.6836x over previous
import functools

import jax
import jax.numpy as jnp
from jax import lax
from jax.experimental import pallas as pl
from jax.experimental.pallas import tpu as pltpu
from jax.experimental.pallas import tpu_sc as plsc

C = 16
H = 1024
W = 1024
N = 1048576
LANES = 16
NUM_CORES = 2
NUM_SUBCORES = 16
NW = NUM_CORES * NUM_SUBCORES
PPW = N // NW
CHUNK = 512
NG16 = CHUNK // LANES
GROWS = 128
NGD = CHUNK // GROWS
NCHUNKS = PPW // CHUNK


def _sc_body(ray_hbm, table_hbm, out_hbm,
             ray_v, i00, i01, i10, i11, w00, w01, w10, w11,
             r00, r01, r10, r11, out_v, sem):
    cid = lax.axis_index("c")
    sid = lax.axis_index("s")
    wid = sid * NUM_CORES + cid
    lane = lax.iota(jnp.int32, LANES)
    zero16 = jnp.zeros((LANES,), jnp.int32)
    one16 = jnp.ones((LANES,), jnp.int32)

    idx_refs = (i00, i01, i10, i11)
    row_refs = (r00, r01, r10, r11)

    def chunk_body(ci, carry):
        base = wid * PPW + ci * CHUNK
        pltpu.sync_copy(ray_hbm.at[pl.ds(base, CHUNK)], ray_v)

        def grp_idx(g, carry2):
            r = g * LANES + lane
            gy = plsc.load_gather(ray_v, [r, zero16])
            gx = plsc.load_gather(ray_v, [r, one16])
            fx = gx * (W - 1.0)
            fy = gy * (H - 1.0)
            fx = jnp.minimum(jnp.maximum(fx, 0.0), W - 1.0)
            fy = jnp.minimum(jnp.maximum(fy, 0.0), H - 1.0)
            x0 = jnp.minimum(fx.astype(jnp.int32), W - 2)
            y0 = jnp.minimum(fy.astype(jnp.int32), H - 2)
            wx1 = fx - x0.astype(jnp.float32)
            wy1 = fy - y0.astype(jnp.float32)
            wx0 = 1.0 - wx1
            wy0 = 1.0 - wy1
            lin = y0 * W + x0
            part = g // (GROWS // LANES)
            off = (g % (GROWS // LANES)) * LANES
            i00[part, pl.ds(off, LANES)] = lin
            i01[part, pl.ds(off, LANES)] = lin + 1
            i10[part, pl.ds(off, LANES)] = lin + W
            i11[part, pl.ds(off, LANES)] = lin + (W + 1)
            s = pl.ds(g * LANES, LANES)
            w00[s] = wy0 * wx0
            w01[s] = wy0 * wx1
            w10[s] = wy1 * wx0
            w11[s] = wy1 * wx1
            return carry2

        lax.fori_loop(0, NG16, grp_idx, 0)

        copies = []
        for cn in range(4):
            for part in range(NGD):
                copies.append(pltpu.async_copy(
                    table_hbm.at[idx_refs[cn].at[part]],
                    row_refs[cn].at[pl.ds(part * GROWS, GROWS)],
                    sem))
        for cp in copies:
            cp.wait()

        def grp_cmb(g, carry2):
            r = g * LANES + lane
            s = pl.ds(g * LANES, LANES)
            a00 = w00[s]
            a01 = w01[s]
            a10 = w10[s]
            a11 = w11[s]
            for ch in range(C):
                cc = jnp.full((LANES,), ch, jnp.int32)
                v = (a00 * plsc.load_gather(r00, [r, cc])
                     + a01 * plsc.load_gather(r01, [r, cc])
                     + a10 * plsc.load_gather(r10, [r, cc])
                     + a11 * plsc.load_gather(r11, [r, cc]))
                plsc.store_scatter(out_v, [r, cc], v)
            return carry2

        lax.fori_loop(0, NG16, grp_cmb, 0)
        pltpu.sync_copy(out_v, out_hbm.at[pl.ds(base, CHUNK)])
        return carry

    lax.fori_loop(0, NCHUNKS, chunk_body, 0)


@functools.partial(jax.jit, static_argnames=())
def _sc_sample(ray, table):
    mesh = plsc.VectorSubcoreMesh(core_axis_name="c", subcore_axis_name="s")
    f = functools.partial(
        pl.kernel, mesh=mesh,
        out_type=jax.ShapeDtypeStruct((N, C), jnp.float32),
        compiler_params=pltpu.CompilerParams(
            needs_layout_passes=False, use_tc_tiling_on_sc=False),
        scratch_types=[
            pltpu.VMEM((CHUNK, 2), jnp.float32),
            pltpu.VMEM((NGD, GROWS), jnp.int32),
            pltpu.VMEM((NGD, GROWS), jnp.int32),
            pltpu.VMEM((NGD, GROWS), jnp.int32),
            pltpu.VMEM((NGD, GROWS), jnp.int32),
            pltpu.VMEM((CHUNK,), jnp.float32),
            pltpu.VMEM((CHUNK,), jnp.float32),
            pltpu.VMEM((CHUNK,), jnp.float32),
            pltpu.VMEM((CHUNK,), jnp.float32),
            pltpu.VMEM((CHUNK, C), jnp.float32),
            pltpu.VMEM((CHUNK, C), jnp.float32),
            pltpu.VMEM((CHUNK, C), jnp.float32),
            pltpu.VMEM((CHUNK, C), jnp.float32),
            pltpu.VMEM((CHUNK, C), jnp.float32),
            pltpu.SemaphoreType.DMA,
        ],
    )(_sc_body)
    return f(ray, table)


def kernel(ray, grid):
    assert ray.shape == (N, 2) and grid.shape == (1, C, H, W)
    table = jnp.swapaxes(grid[0].reshape(C, H * W), 0, 1)
    return _sc_sample(ray, table)
